# baseline (speedup 1.0000x reference)
import functools

import jax
import jax.numpy as jnp
from jax import lax
from jax.experimental import pallas as pl
from jax.experimental.pallas import tpu as pltpu
from jax.experimental.pallas import tpu_sc as plsc

_B, _S, _D = 4, 2048, 128
_N = _B * _S
_NW = 32
_ROWS = _N // _NW
_NB = _D // 16
_NCH = 2
_CH = _ROWS // _NCH


def _embed_sum(tok_idx2, seg_f32, token_table, seg_table, pe2d):
    mesh = plsc.VectorSubcoreMesh(core_axis_name="c", subcore_axis_name="s")

    @functools.partial(
        pl.kernel,
        out_type=jax.ShapeDtypeStruct((_N, _D), jnp.float32),
        mesh=mesh,
        scratch_types=[
            pltpu.VMEM((_NCH, _CH), jnp.int32),
            pltpu.VMEM((_ROWS,), jnp.float32),
            pltpu.VMEM((_ROWS, _D), jnp.float32),
            pltpu.VMEM((2, _D), jnp.float32),
            pltpu.VMEM((_ROWS, _D), jnp.float32),
            [pltpu.SemaphoreType.DMA] * _NCH,
            [pltpu.SemaphoreType.DMA] * _NCH,
            pltpu.SemaphoreType.DMA,
            pltpu.SemaphoreType.DMA,
            pltpu.SemaphoreType.DMA,
        ],
    )
    def k(tok_hbm, seg_hbm, table_hbm, segtab_hbm, pe_hbm, out_hbm,
          tok_v, seg_v, pe_v, st_v, acc_v, gsems, wsems, psem, ssem, lsem):
        wid = lax.axis_index("s") * 2 + lax.axis_index("c")
        base = wid * _ROWS
        pbase = lax.rem(base, _S)
        c_pe = pltpu.async_copy(pe_hbm.at[pl.ds(pbase, _ROWS)], pe_v, psem)
        c_st = pltpu.async_copy(segtab_hbm, st_v, ssem)
        c_lb = pltpu.async_copy(seg_hbm.at[wid], seg_v, lsem)
        pltpu.sync_copy(tok_hbm.at[wid], tok_v)
        gathers = [
            pltpu.async_copy(table_hbm.at[tok_v.at[j]],
                             acc_v.at[pl.ds(j * _CH, _CH)], gsems[j])
            for j in range(_NCH)
        ]
        c_st.wait()
        c_lb.wait()
        c_pe.wait()
        st0 = [st_v[0, pl.ds(16 * j, 16)] for j in range(_NB)]
        dif = [st_v[1, pl.ds(16 * j, 16)] - st0[j] for j in range(_NB)]

        writes = []
        for ch in range(_NCH):
            gathers[ch].wait()

            @plsc.parallel_loop(ch * _CH, (ch + 1) * _CH, 16, unroll=1)
            def _merge(i0):
                lbl16 = seg_v[pl.ds(i0, 16)]
                for t in range(16):
                    lbl = jnp.take_along_axis(
                        lbl16, jnp.full((16,), t, jnp.int32),
                        axis=0, mode="promise_in_bounds")
                    for j in range(_NB):
                        sl = (i0 + t, pl.ds(16 * j, 16))
                        acc_v[sl] = (acc_v[sl] + pe_v[sl]
                                     + (st0[j] + lbl * dif[j]))

            writes.append(pltpu.async_copy(
                acc_v.at[pl.ds(ch * _CH, _CH)],
                out_hbm.at[pl.ds(base + ch * _CH, _CH)], wsems[ch]))
        for w in writes:
            w.wait()

    return k(tok_idx2, seg_f32, token_table, seg_table, pe2d)


def kernel(tok_idx, segment_label, token_table, seg_table, pe):
    tok_idx2 = tok_idx.reshape(_NW, _NCH, _CH).astype(jnp.int32)
    seg_f32 = segment_label.reshape(_NW, _ROWS).astype(jnp.float32)
    pe2d = pe.reshape(_S, _D)
    out = _embed_sum(tok_idx2, seg_f32, token_table, seg_table, pe2d)
    return out.reshape(_B, _S, _D)


# speedup vs baseline: 1.2513x; 1.2513x over previous
import functools

import jax
import jax.numpy as jnp
from jax import lax
from jax.experimental import pallas as pl
from jax.experimental.pallas import tpu as pltpu
from jax.experimental.pallas import tpu_sc as plsc

_B, _S, _D = 4, 2048, 128
_N = _B * _S
_NW = 32
_ROWS = _N // _NW
_NB = _D // 16
_NCH = 2
_CH = _ROWS // _NCH


def _embed_sum(tok_idx2, seg_f32, token_table, seg_table, pe2d):
    mesh = plsc.VectorSubcoreMesh(core_axis_name="c", subcore_axis_name="s")

    @functools.partial(
        pl.kernel,
        out_type=jax.ShapeDtypeStruct((_N, _D), jnp.float32),
        mesh=mesh,
        scratch_types=[
            pltpu.VMEM((_NCH, _CH), jnp.int32),
            pltpu.VMEM((_ROWS,), jnp.float32),
            pltpu.VMEM((_ROWS, _D), jnp.float32),
            pltpu.VMEM((2, _D), jnp.float32),
            pltpu.VMEM((_ROWS, _D), jnp.float32),
            [pltpu.SemaphoreType.DMA] * _NCH,
            [pltpu.SemaphoreType.DMA] * _NCH,
            pltpu.SemaphoreType.DMA,
            pltpu.SemaphoreType.DMA,
            pltpu.SemaphoreType.DMA,
        ],
    )
    def k(tok_hbm, seg_hbm, table_hbm, segtab_hbm, pe_hbm, out_hbm,
          tok_v, seg_v, pe_v, st_v, acc_v, gsems, wsems, psem, ssem, lsem):
        wid = lax.axis_index("s") * 2 + lax.axis_index("c")
        base = wid * _ROWS
        pbase = lax.rem(base, _S)
        c_pe = pltpu.async_copy(pe_hbm.at[pl.ds(pbase, _ROWS)], pe_v, psem)
        c_st = pltpu.async_copy(segtab_hbm, st_v, ssem)
        c_lb = pltpu.async_copy(seg_hbm.at[wid], seg_v, lsem)
        pltpu.sync_copy(tok_hbm.at[wid], tok_v)
        gathers = [
            pltpu.async_copy(table_hbm.at[tok_v.at[j]],
                             acc_v.at[pl.ds(j * _CH, _CH)], gsems[j])
            for j in range(_NCH)
        ]
        c_st.wait()
        c_lb.wait()
        c_pe.wait()
        st0 = [st_v[0, pl.ds(16 * j, 16)] for j in range(_NB)]
        dif = [st_v[1, pl.ds(16 * j, 16)] - st0[j] for j in range(_NB)]

        writes = []
        for ch in range(_NCH):
            gathers[ch].wait()

            @plsc.parallel_loop(ch * _CH, (ch + 1) * _CH, 1, unroll=8)
            def _merge(i):
                lbl16 = seg_v[pl.ds(16 * lax.div(i, 16), 16)]
                lbl = jnp.take_along_axis(
                    lbl16, jnp.full((16,), lax.rem(i, 16), jnp.int32),
                    axis=0, mode="promise_in_bounds")
                for j in range(_NB):
                    sl = (i, pl.ds(16 * j, 16))
                    plsc.addupdate(pe_v.at[sl],
                                   acc_v[sl] + (st0[j] + lbl * dif[j]))

            writes.append(pltpu.async_copy(
                pe_v.at[pl.ds(ch * _CH, _CH)],
                out_hbm.at[pl.ds(base + ch * _CH, _CH)], wsems[ch]))
        for w in writes:
            w.wait()

    return k(tok_idx2, seg_f32, token_table, seg_table, pe2d)


def kernel(tok_idx, segment_label, token_table, seg_table, pe):
    tok_idx2 = tok_idx.reshape(_NW, _NCH, _CH).astype(jnp.int32)
    seg_f32 = segment_label.reshape(_NW, _ROWS).astype(jnp.float32)
    pe2d = pe.reshape(_S, _D)
    out = _embed_sum(tok_idx2, seg_f32, token_table, seg_table, pe2d)
    return out.reshape(_B, _S, _D)


# speedup vs baseline: 1.4266x; 1.1401x over previous
import functools

import jax
import jax.numpy as jnp
from jax import lax
from jax.experimental import pallas as pl
from jax.experimental.pallas import tpu as pltpu
from jax.experimental.pallas import tpu_sc as plsc

_B, _S, _D = 4, 2048, 128
_N = _B * _S
_NW = 32
_ROWS = _N // _NW
_NB = _D // 16
_NCH = 2
_CH = _ROWS // _NCH


def _embed_sum(tok_idx2, seg_f32, token_table, seg_table, pe2d):
    mesh = plsc.VectorSubcoreMesh(core_axis_name="c", subcore_axis_name="s")

    @functools.partial(
        pl.kernel,
        out_type=jax.ShapeDtypeStruct((_N, _D), jnp.float32),
        mesh=mesh,
        scratch_types=[
            pltpu.VMEM((_NCH, _CH), jnp.int32),
            pltpu.VMEM((_ROWS,), jnp.float32),
            pltpu.VMEM((_ROWS, _D), jnp.float32),
            pltpu.VMEM((2, _D), jnp.float32),
            pltpu.VMEM((_ROWS, _D), jnp.float32),
            [pltpu.SemaphoreType.DMA] * _NCH,
            [pltpu.SemaphoreType.DMA] * _NCH,
            pltpu.SemaphoreType.DMA,
            pltpu.SemaphoreType.DMA,
            pltpu.SemaphoreType.DMA,
        ],
    )
    def k(tok_hbm, seg_hbm, table_hbm, segtab_hbm, pe_hbm, out_hbm,
          tok_v, seg_v, pe_v, st_v, acc_v, gsems, wsems, psem, ssem, lsem):
        wid = lax.axis_index("s") * 2 + lax.axis_index("c")
        base = wid * _ROWS
        pbase = lax.rem(base, _S)
        c_pe = pltpu.async_copy(pe_hbm.at[pl.ds(pbase, _ROWS)], pe_v, psem)
        c_st = pltpu.async_copy(segtab_hbm, st_v, ssem)
        c_lb = pltpu.async_copy(seg_hbm.at[wid], seg_v, lsem)
        pltpu.sync_copy(tok_hbm.at[wid], tok_v)
        gathers = [
            pltpu.async_copy(table_hbm.at[tok_v.at[j]],
                             acc_v.at[pl.ds(j * _CH, _CH)], gsems[j])
            for j in range(_NCH)
        ]
        c_st.wait()
        c_lb.wait()
        c_pe.wait()
        st0 = [st_v[0, pl.ds(16 * j, 16)] for j in range(_NB)]
        dif = [st_v[1, pl.ds(16 * j, 16)] - st0[j] for j in range(_NB)]

        writes = []
        for ch in range(_NCH):
            gathers[ch].wait()

            @plsc.parallel_loop(ch * _CH, (ch + 1) * _CH, 1, unroll=8)
            def _merge(i):
                lbl16 = seg_v[pl.ds(16 * lax.div(i, 16), 16)]
                lbl = jnp.take_along_axis(
                    lbl16, jnp.full((16,), lax.rem(i, 16), jnp.int32),
                    axis=0, mode="promise_in_bounds")
                for j in range(_NB):
                    sl = (i, pl.ds(16 * j, 16))
                    acc_v[sl] = acc_v[sl] + pe_v[sl] + (st0[j] + lbl * dif[j])

            writes.append(pltpu.async_copy(
                acc_v.at[pl.ds(ch * _CH, _CH)],
                out_hbm.at[pl.ds(base + ch * _CH, _CH)], wsems[ch]))
        for w in writes:
            w.wait()

    return k(tok_idx2, seg_f32, token_table, seg_table, pe2d)


def kernel(tok_idx, segment_label, token_table, seg_table, pe):
    tok_idx2 = tok_idx.reshape(_NW, _NCH, _CH).astype(jnp.int32)
    seg_f32 = segment_label.reshape(_NW, _ROWS).astype(jnp.float32)
    pe2d = pe.reshape(_S, _D)
    out = _embed_sum(tok_idx2, seg_f32, token_table, seg_table, pe2d)
    return out.reshape(_B, _S, _D)
